# Initial kernel scaffold; baseline (speedup 1.0000x reference)
import jax
import jax.numpy as jnp
from jax.experimental import pallas as pl


def kernel(x, edge_index, W_m, a_src_m, a_dst_m, b_m, W_s, a_src_s, a_dst_s, b_s):
    raise NotImplementedError("write your pallas kernel here")


# speedup vs baseline: 29.7111x; 29.7111x over previous
import jax
import jax.numpy as jnp
from jax import lax
from jax.experimental import pallas as pl
from jax.experimental.pallas import tpu as pltpu
from jax.experimental.pallas import tpu_sc as plsc

NC = 2
NS = 16
L = 16
NW = NC * NS
H = 4
KA = 80
KB = 80


def _tc1_body(x_ref, wm_ref, asm_ref, adm_ref, ws_ref, ass_ref, ads_ref,
              hm_ref, hbm_ref, tm_ref, am_ref, bm_ref,
              hs_ref, hbs_ref, ts_ref, as_ref, ad_ref):
    xb = x_ref[...]
    bn = xb.shape[0]
    zpad = jnp.zeros((bn, 2 * H), jnp.float32)
    for w_r, asrc_r, adst_r, h_r, hb_r, t_r, ao_r, bo_r in (
        (wm_ref, asm_ref, adm_ref, hm_ref, hbm_ref, tm_ref, am_ref, bm_ref),
        (ws_ref, ass_ref, ads_ref, hs_ref, hbs_ref, ts_ref, as_ref, ad_ref),
    ):
        h = jnp.dot(xb, w_r[...], preferred_element_type=jnp.float32)
        h_r[...] = h
        hb_r[...] = h.astype(jnp.bfloat16)
        h3 = h.reshape(bn, H, h.shape[1] // H)
        a = (h3 * asrc_r[...][None]).sum(-1)
        b = (h3 * adst_r[...][None]).sum(-1)
        ao_r[...] = a
        bo_r[...] = b
        t_r[...] = jnp.concatenate([a, b, zpad], axis=1)


def _tc2_body(dpm_ref, am_ref, bm_ref, hm_ref, dps_ref, as_ref, ad_ref, hs_ref,
              dim_ref, lm_ref, dis_ref, ls_ref):
    for dp_r, a_r, b_r, h_r, di_r, lo_r in (
        (dpm_ref, am_ref, bm_ref, hm_ref, dim_ref, lm_ref),
        (dps_ref, as_ref, ad_ref, hs_ref, dis_ref, ls_ref),
    ):
        logit = a_r[...] + b_r[...]
        logit = jnp.where(logit >= 0, logit, 0.2 * logit)
        num_loop = jnp.exp(logit)
        dp = dp_r[...]
        den = dp[0, :, :H] + dp[1, :, :H] + num_loop
        dinv = 1.0 / (den + 1e-16)
        bn = dinv.shape[0]
        di_r[...] = jnp.concatenate(
            [dinv, jnp.zeros((bn, 16 - H), jnp.float32)], axis=1)
        w = num_loop * dinv
        hb = h_r[...]
        h3 = hb.reshape(bn, H, hb.shape[1] // H)
        lo_r[...] = (h3 * w[:, :, None]).sum(axis=1)


def _tc3_body(opm_ref, lm_ref, bm_ref, ops_ref, ls_ref, bs_ref,
              zm_ref, zs_ref):
    for op_r, lo_r, b_r, z_r in ((opm_ref, lm_ref, bm_ref, zm_ref),
                                 (ops_ref, ls_ref, bs_ref, zs_ref)):
        op = op_r[...]
        z_r[...] = (op[0] + op[1] + lo_r[...]) * (1.0 / H) + b_r[...]


def _sc_a_body(src_h, dst_h, tm_h, ts_h, zer_h,
               numm_h, dnpm_h, nums_h, dnps_h,
               asb_v, adb_v, src_v, dst_v, num4_v, num16_v, den_sh, sem):
    c = lax.axis_index("c")
    s = lax.axis_index("s")
    wid = s * NC + c
    ept = src_h.shape[0] // NW
    zv = jnp.zeros((L,), jnp.float32)
    for r in range(KA):
        num16_v[r, :] = zv
    iota = lax.iota(jnp.int32, L)

    for t_h, num_h, dnp_h in ((tm_h, numm_h, dnpm_h), (ts_h, nums_h, dnps_h)):
        @pl.when(s == 0)
        def _():
            pltpu.sync_copy(zer_h, den_sh)
        plsc.subcore_barrier()

        def chunk(ci, carry):
            cb = wid * ept + ci * KA
            pltpu.sync_copy(src_h.at[pl.ds(cb, KA)], src_v)
            pltpu.sync_copy(dst_h.at[pl.ds(cb, KA)], dst_v)
            pltpu.async_copy(t_h.at[src_v], asb_v, sem).wait()
            pltpu.async_copy(t_h.at[dst_v], adb_v, sem).wait()

            def grp(g, carry2):
                row = iota + g * L
                for h in range(H):
                    hh = jnp.full((L,), h, jnp.int32)
                    lo = (plsc.load_gather(asb_v, [row, hh])
                          + plsc.load_gather(adb_v, [row, hh + H]))
                    lo = jnp.where(lo >= 0, lo, 0.2 * lo)
                    nm = jnp.exp(lo)
                    plsc.store_scatter(num4_v, [row, hh], nm)
                    plsc.store_scatter(num16_v, [row, hh], nm)
                return carry2

            lax.fori_loop(0, KA // L, grp, 0)
            pltpu.sync_copy(num4_v, num_h.at[pl.ds(cb, KA)])
            pltpu.sync_copy(num16_v, den_sh.at[dst_v], add=True)
            return carry

        lax.fori_loop(0, ept // KA, chunk, 0)
        plsc.subcore_barrier()
        @pl.when(s == 0)
        def _():
            pltpu.sync_copy(den_sh, dnp_h.at[c])
        plsc.subcore_barrier()


def _sc_b_body(src_h, dst_h, numm_h, dinvm_h, hm_h, nums_h, dinvs_h, hs_h,
               zer_h, outpm_h, outps_h,
               src_v, dst_v, num_v, dib_v, w_v, hbuf_v, msg_v, out_sh, sem):
    c = lax.axis_index("c")
    s = lax.axis_index("s")
    wid = s * NC + c
    ept = src_h.shape[0] // NW
    hc = hm_h.shape[1]
    cdim = hc // H
    iota = lax.iota(jnp.int32, L)

    for num_h, dinv_h, hrows_h, outp_h in (
        (numm_h, dinvm_h, hm_h, outpm_h),
        (nums_h, dinvs_h, hs_h, outps_h),
    ):
        @pl.when(s == 0)
        def _():
            pltpu.sync_copy(zer_h, out_sh)
        plsc.subcore_barrier()

        def chunk(ci, carry):
            cb = wid * ept + ci * KB
            pltpu.sync_copy(src_h.at[pl.ds(cb, KB)], src_v)
            pltpu.sync_copy(dst_h.at[pl.ds(cb, KB)], dst_v)
            pltpu.sync_copy(num_h.at[pl.ds(cb, KB)], num_v)
            pltpu.async_copy(dinv_h.at[dst_v], dib_v, sem).wait()
            pltpu.async_copy(hrows_h.at[src_v], hbuf_v, sem).wait()

            def grp(g, carry2):
                row = iota + g * L
                for h in range(H):
                    hh = jnp.full((L,), h, jnp.int32)
                    di = plsc.load_gather(dib_v, [row, hh])
                    nm = plsc.load_gather(num_v, [row, hh])
                    plsc.store_scatter(w_v, [row, hh], nm * di)
                return carry2

            lax.fori_loop(0, KB // L, grp, 0)

            def edge(k, carry2):
                zi = jnp.zeros((L,), jnp.int32)
                kk = zi + k
                wk = [plsc.load_gather(w_v, [kk, zi + h]) for h in range(H)]
                for j in range(cdim // (2 * L)):
                    acc_e = None
                    acc_o = None
                    for h in range(H):
                        hb = hbuf_v[k, pl.ds(h * cdim + 2 * L * j, 2 * L)]
                        ev, od = plsc.unpack(
                            hb, format=plsc.PackFormat.INTERLEAVED)
                        if acc_e is None:
                            acc_e = wk[h] * ev
                            acc_o = wk[h] * od
                        else:
                            acc_e = acc_e + wk[h] * ev
                            acc_o = acc_o + wk[h] * od
                    ce = 2 * iota + 2 * L * j
                    plsc.store_scatter(msg_v, [kk, ce], acc_e)
                    plsc.store_scatter(msg_v, [kk, ce + 1], acc_o)
                return carry2

            lax.fori_loop(0, KB, edge, 0)
            pltpu.sync_copy(msg_v, out_sh.at[dst_v], add=True)
            return carry

        lax.fori_loop(0, ept // KB, chunk, 0)
        plsc.subcore_barrier()
        @pl.when(s == 0)
        def _():
            pltpu.sync_copy(out_sh, outp_h.at[c])
        plsc.subcore_barrier()


def kernel(x, edge_index, W_m, a_src_m, a_dst_m, b_m, W_s, a_src_s, a_dst_s, b_s):
    n, d = x.shape
    hc = W_m.shape[1]
    cdim = hc // H
    e = edge_index.shape[1]
    src = edge_index[0]
    dst = edge_index[1]

    bn = 2000
    grid = (n // bn,)
    f32 = jnp.float32

    tc1 = pl.pallas_call(
        _tc1_body,
        grid=grid,
        in_specs=[
            pl.BlockSpec((bn, d), lambda i: (i, 0)),
            pl.BlockSpec((d, hc), lambda i: (0, 0)),
            pl.BlockSpec((H, cdim), lambda i: (0, 0)),
            pl.BlockSpec((H, cdim), lambda i: (0, 0)),
            pl.BlockSpec((d, hc), lambda i: (0, 0)),
            pl.BlockSpec((H, cdim), lambda i: (0, 0)),
            pl.BlockSpec((H, cdim), lambda i: (0, 0)),
        ],
        out_specs=[
            pl.BlockSpec((bn, hc), lambda i: (i, 0)),
            pl.BlockSpec((bn, hc), lambda i: (i, 0)),
            pl.BlockSpec((bn, 16), lambda i: (i, 0)),
            pl.BlockSpec((bn, H), lambda i: (i, 0)),
            pl.BlockSpec((bn, H), lambda i: (i, 0)),
            pl.BlockSpec((bn, hc), lambda i: (i, 0)),
            pl.BlockSpec((bn, hc), lambda i: (i, 0)),
            pl.BlockSpec((bn, 16), lambda i: (i, 0)),
            pl.BlockSpec((bn, H), lambda i: (i, 0)),
            pl.BlockSpec((bn, H), lambda i: (i, 0)),
        ],
        out_shape=[
            jax.ShapeDtypeStruct((n, hc), f32),
            jax.ShapeDtypeStruct((n, hc), jnp.bfloat16),
            jax.ShapeDtypeStruct((n, 16), f32),
            jax.ShapeDtypeStruct((n, H), f32),
            jax.ShapeDtypeStruct((n, H), f32),
            jax.ShapeDtypeStruct((n, hc), f32),
            jax.ShapeDtypeStruct((n, hc), jnp.bfloat16),
            jax.ShapeDtypeStruct((n, 16), f32),
            jax.ShapeDtypeStruct((n, H), f32),
            jax.ShapeDtypeStruct((n, H), f32),
        ],
    )
    (h_m, hb_m, t_m, as_m, ad_m,
     h_s, hb_s, t_s, as_s, ad_s) = tc1(x, W_m, a_src_m, a_dst_m,
                                       W_s, a_src_s, a_dst_s)

    mesh = plsc.VectorSubcoreMesh(core_axis_name="c", subcore_axis_name="s",
                                  num_cores=NC, num_subcores=NS)
    sc_params = pltpu.CompilerParams(needs_layout_passes=False,
                                     use_tc_tiling_on_sc=False)
    zeros_d = jnp.zeros((n, 16), f32)
    zeros_c = jnp.zeros((n, cdim), f32)

    sc_a = pl.kernel(
        _sc_a_body,
        out_type=[
            jax.ShapeDtypeStruct((e, H), f32),
            jax.ShapeDtypeStruct((NC, n, 16), f32),
            jax.ShapeDtypeStruct((e, H), f32),
            jax.ShapeDtypeStruct((NC, n, 16), f32),
        ],
        mesh=mesh,
        scratch_types=[
            pltpu.VMEM((KA, 16), f32),
            pltpu.VMEM((KA, 16), f32),
            pltpu.VMEM((KA,), jnp.int32),
            pltpu.VMEM((KA,), jnp.int32),
            pltpu.VMEM((KA, H), f32),
            pltpu.VMEM((KA, 16), f32),
            pltpu.VMEM_SHARED((n, 16), f32),
            pltpu.SemaphoreType.DMA,
        ],
        compiler_params=sc_params,
    )

    num_m, denp_m, num_s, denp_s = sc_a(src, dst, t_m, t_s, zeros_d)

    tc2 = pl.pallas_call(
        _tc2_body,
        grid=grid,
        in_specs=[
            pl.BlockSpec((NC, bn, 16), lambda i: (0, i, 0)),
            pl.BlockSpec((bn, H), lambda i: (i, 0)),
            pl.BlockSpec((bn, H), lambda i: (i, 0)),
            pl.BlockSpec((bn, hc), lambda i: (i, 0)),
            pl.BlockSpec((NC, bn, 16), lambda i: (0, i, 0)),
            pl.BlockSpec((bn, H), lambda i: (i, 0)),
            pl.BlockSpec((bn, H), lambda i: (i, 0)),
            pl.BlockSpec((bn, hc), lambda i: (i, 0)),
        ],
        out_specs=[
            pl.BlockSpec((bn, 16), lambda i: (i, 0)),
            pl.BlockSpec((bn, cdim), lambda i: (i, 0)),
            pl.BlockSpec((bn, 16), lambda i: (i, 0)),
            pl.BlockSpec((bn, cdim), lambda i: (i, 0)),
        ],
        out_shape=[
            jax.ShapeDtypeStruct((n, 16), f32),
            jax.ShapeDtypeStruct((n, cdim), f32),
            jax.ShapeDtypeStruct((n, 16), f32),
            jax.ShapeDtypeStruct((n, cdim), f32),
        ],
    )
    dinv_m, loop_m, dinv_s, loop_s = tc2(denp_m, as_m, ad_m, h_m,
                                         denp_s, as_s, ad_s, h_s)

    sc_b = pl.kernel(
        _sc_b_body,
        out_type=[
            jax.ShapeDtypeStruct((NC, n, cdim), f32),
            jax.ShapeDtypeStruct((NC, n, cdim), f32),
        ],
        mesh=mesh,
        scratch_types=[
            pltpu.VMEM((KB,), jnp.int32),
            pltpu.VMEM((KB,), jnp.int32),
            pltpu.VMEM((KB, H), f32),
            pltpu.VMEM((KB, 16), f32),
            pltpu.VMEM((KB, H), f32),
            pltpu.VMEM((KB, hc), jnp.bfloat16),
            pltpu.VMEM((KB, cdim), f32),
            pltpu.VMEM_SHARED((n, cdim), f32),
            pltpu.SemaphoreType.DMA,
        ],
        compiler_params=sc_params,
    )

    outp_m, outp_s = sc_b(src, dst, num_m, dinv_m, hb_m,
                          num_s, dinv_s, hb_s, zeros_c)

    tc3 = pl.pallas_call(
        _tc3_body,
        grid=grid,
        in_specs=[
            pl.BlockSpec((NC, bn, cdim), lambda i: (0, i, 0)),
            pl.BlockSpec((bn, cdim), lambda i: (i, 0)),
            pl.BlockSpec((1, cdim), lambda i: (0, 0)),
            pl.BlockSpec((NC, bn, cdim), lambda i: (0, i, 0)),
            pl.BlockSpec((bn, cdim), lambda i: (i, 0)),
            pl.BlockSpec((1, cdim), lambda i: (0, 0)),
        ],
        out_specs=[
            pl.BlockSpec((bn, cdim), lambda i: (i, 0)),
            pl.BlockSpec((bn, cdim), lambda i: (i, 0)),
        ],
        out_shape=[
            jax.ShapeDtypeStruct((n, cdim), f32),
            jax.ShapeDtypeStruct((n, cdim), f32),
        ],
    )
    z_m, z_s = tc3(outp_m, loop_m, b_m.reshape(1, cdim),
                   outp_s, loop_s, b_s.reshape(1, cdim))
    return (z_m, z_s)
